# baseline (speedup 1.0000x reference)
import functools

import jax
import jax.numpy as jnp
from jax import lax
from jax.experimental import pallas as pl
from jax.experimental.pallas import tpu as pltpu
from jax.experimental.pallas import tpu_sc as plsc

EPS = 1e-3


def _make_sc_gather(V, D, N, CH=512):
    info = plsc.get_sparse_core_info()
    NC, NS = info.num_cores, info.num_subcores
    NW = NC * NS
    n_per_w = N // NW
    n_chunks = n_per_w // CH
    assert n_per_w % CH == 0

    mesh = plsc.VectorSubcoreMesh(core_axis_name="c", subcore_axis_name="s")

    @functools.partial(
        pl.kernel,
        mesh=mesh,
        out_type=jax.ShapeDtypeStruct((N, D), jnp.float32),
        scratch_types=[
            pltpu.VMEM((CH,), jnp.int32),
            pltpu.VMEM((CH, D), jnp.float32),
            pltpu.SemaphoreType.DMA,
        ],
    )
    def gather_k(idx_hbm, table_hbm, out_hbm, idx_v, rows_v, sem):
        wid = lax.axis_index("s") * NC + lax.axis_index("c")
        base = wid * n_per_w

        def body(i, carry):
            off = base + i * CH
            pltpu.sync_copy(idx_hbm.at[pl.ds(off, CH)], idx_v)
            pltpu.async_copy(table_hbm.at[idx_v], rows_v, sem).wait()
            pltpu.sync_copy(rows_v, out_hbm.at[pl.ds(off, CH)])
            return carry

        lax.fori_loop(0, n_chunks, body, 0)

    return gather_k


def _ln_body(sum_ref, pos_ref, tt_ref, type_ref, gamma_ref, beta_ref, out_ref):
    x = sum_ref[...] + pos_ref[...]
    ttf = tt_ref[...]
    t0 = type_ref[0:1, :]
    t1 = type_ref[1:2, :]
    x = x + t0 + ttf * (t1 - t0)
    d = x.shape[1]
    ones = jnp.ones((d, 8), jnp.float32)
    s1 = lax.dot_general(x, ones, (((1,), (0,)), ((), ())),
                         preferred_element_type=jnp.float32)[:, 0:1]
    s2 = lax.dot_general(x * x, ones, (((1,), (0,)), ((), ())),
                         preferred_element_type=jnp.float32)[:, 0:1]
    mean = s1 * (1.0 / d)
    var = s2 * (1.0 / d) - mean * mean
    y = (x - mean) * lax.rsqrt(var + EPS)
    out_ref[...] = y * gamma_ref[...] + beta_ref[...]


def _ln_call(summed, position_table, tt3, type_table, gamma2, beta2, B, S, D):
    return pl.pallas_call(
        _ln_body,
        grid=(B,),
        in_specs=[
            pl.BlockSpec((S, D), lambda i: (i, 0)),
            pl.BlockSpec((S, D), lambda i: (0, 0)),
            pl.BlockSpec((S, 1), lambda i: (i, 0)),
            pl.BlockSpec((2, D), lambda i: (0, 0)),
            pl.BlockSpec((1, D), lambda i: (0, 0)),
            pl.BlockSpec((1, D), lambda i: (0, 0)),
        ],
        out_specs=pl.BlockSpec((S, D), lambda i: (i, 0)),
        out_shape=jax.ShapeDtypeStruct((B * S, D), jnp.float32),
    )(summed, position_table, tt3, type_table, gamma2, beta2)


def kernel(input_ids, token_type_ids, token_embedding, position_table, type_table, gamma, beta):
    B, S = input_ids.shape
    V, D = token_embedding.shape
    N = B * S

    idx_flat = input_ids.reshape(N).astype(jnp.int32)
    gathered = _make_sc_gather(V, D, N)(idx_flat, token_embedding)

    ttf = token_type_ids.reshape(N, 1).astype(jnp.float32)
    out = _ln_call(
        gathered, position_table, ttf, type_table,
        gamma.reshape(1, D), beta.reshape(1, D), B, S, D,
    )
    return out.reshape(B, S, D), token_embedding


# speedup vs baseline: 1.6449x; 1.6449x over previous
import functools

import jax
import jax.numpy as jnp
from jax import lax
from jax.experimental import pallas as pl
from jax.experimental.pallas import tpu as pltpu
from jax.experimental.pallas import tpu_sc as plsc

EPS = 1e-3


def _make_sc_gather(V, D, N, CH=512):
    info = plsc.get_sparse_core_info()
    NC, NS = info.num_cores, info.num_subcores
    NW = NC * NS
    n_per_w = N // NW
    n_chunks = n_per_w // CH
    assert n_per_w % CH == 0

    mesh = plsc.VectorSubcoreMesh(core_axis_name="c", subcore_axis_name="s")

    @functools.partial(
        pl.kernel,
        mesh=mesh,
        out_type=jax.ShapeDtypeStruct((N, D), jnp.float32),
        scratch_types=[
            pltpu.VMEM((CH,), jnp.int32),
            pltpu.VMEM((CH, D), jnp.float32),
            pltpu.SemaphoreType.DMA,
        ],
    )
    def gather_k(idx_hbm, table_hbm, out_hbm, idx_v, rows_v, sem):
        wid = lax.axis_index("s") * NC + lax.axis_index("c")
        base = wid * n_per_w

        def body(i, carry):
            off = base + i * CH
            pltpu.sync_copy(idx_hbm.at[pl.ds(off, CH)], idx_v)
            pltpu.async_copy(table_hbm.at[idx_v], rows_v, sem).wait()
            pltpu.sync_copy(rows_v, out_hbm.at[pl.ds(off, CH)])
            return carry

        lax.fori_loop(0, n_chunks, body, 0)

    return gather_k


def _ln_body(sum_ref, pos_ref, tt_ref, type_ref, gamma_ref, beta_ref, out_ref):
    x = sum_ref[...] + pos_ref[...]
    ttf = tt_ref[...]
    t0 = type_ref[0:1, :]
    t1 = type_ref[1:2, :]
    x = x + t0 + ttf * (t1 - t0)
    mean = jnp.mean(x, axis=-1, keepdims=True)
    xc = x - mean
    var = jnp.mean(xc * xc, axis=-1, keepdims=True)
    y = xc * lax.rsqrt(var + EPS)
    out_ref[...] = y * gamma_ref[...] + beta_ref[...]


def _ln_call(summed, pos_tiled, ttf, type_table, gamma2, beta2, B, S, D, K=4):
    T = K * S
    return pl.pallas_call(
        _ln_body,
        grid=(B // K,),
        in_specs=[
            pl.BlockSpec((T, D), lambda i: (i, 0)),
            pl.BlockSpec((T, D), lambda i: (0, 0)),
            pl.BlockSpec((T, 1), lambda i: (i, 0)),
            pl.BlockSpec((2, D), lambda i: (0, 0)),
            pl.BlockSpec((1, D), lambda i: (0, 0)),
            pl.BlockSpec((1, D), lambda i: (0, 0)),
        ],
        out_specs=pl.BlockSpec((T, D), lambda i: (i, 0)),
        out_shape=jax.ShapeDtypeStruct((B * S, D), jnp.float32),
    )(summed, pos_tiled, ttf, type_table, gamma2, beta2)


def kernel(input_ids, token_type_ids, token_embedding, position_table, type_table, gamma, beta):
    B, S = input_ids.shape
    V, D = token_embedding.shape
    N = B * S

    idx_flat = input_ids.reshape(N).astype(jnp.int32)
    gathered = _make_sc_gather(V, D, N)(idx_flat, token_embedding)

    ttf = token_type_ids.reshape(N, 1).astype(jnp.float32)
    pos_tiled = jnp.tile(position_table, (4, 1))
    out = _ln_call(
        gathered, pos_tiled, ttf, type_table,
        gamma.reshape(1, D), beta.reshape(1, D), B, S, D, K=4,
    )
    return out.reshape(B, S, D), token_embedding
